# Initial kernel scaffold; baseline (speedup 1.0000x reference)
import jax
import jax.numpy as jnp
from jax.experimental import pallas as pl


def kernel(hm, reg, wh, node, fmap):
    raise NotImplementedError("write your pallas kernel here")


# speedup vs baseline: 1.1150x; 1.1150x over previous
import functools

import jax
import jax.numpy as jnp
from jax import lax
from jax.experimental import pallas as pl
from jax.experimental.pallas import tpu as pltpu
from jax.experimental.pallas import tpu_sc as plsc

_B, _H, _W = 4, 152, 272
_HW = _H * _W
_K = 500
_NC = 512
_D = 384
_R, _L = 323, 128
_CR = 19
_NCH = _R // _CR
_CW = _CR * _L

_F32 = jnp.float32
_HI = jax.lax.Precision.HIGHEST


def _nms_kernel(hm_ref, out_ref):
    neg = _F32(-1e30)
    s = jax.nn.sigmoid(hm_ref[0, 0])
    colpad = jnp.full((_H, 1), neg, _F32)
    h = jnp.maximum(
        jnp.maximum(jnp.concatenate([s[:, 1:], colpad], axis=1),
                    jnp.concatenate([colpad, s[:, :-1]], axis=1)), s)
    rowpad = jnp.full((1, _W), neg, _F32)
    m = jnp.maximum(
        jnp.maximum(jnp.concatenate([h[1:, :], rowpad], axis=0),
                    jnp.concatenate([rowpad, h[:-1, :]], axis=0)), h)
    out_ref[0, 0] = jnp.where(m == s, s, _F32(0.0))


def _topk_kernel(s_ref, reg_ref, wh_ref,
                 scores_ref, idx_ref, tlbr_ref, xs_ref, ys_ref, wh2_ref):
    one = _F32(1.0)
    zero = _F32(0.0)
    ssup = s_ref[0, 0]

    bits = lax.bitcast_convert_type(ssup, jnp.int32)

    def bis(_, carry):
        lo, hi = carry
        mid = lo + (hi - lo) // 2
        cnt = jnp.sum(jnp.where(bits >= mid, one, zero))
        ok = cnt >= _F32(_K)
        return jnp.where(ok, mid, lo), jnp.where(ok, hi, mid)

    tbits, _ = lax.fori_loop(0, 31, bis, (jnp.int32(0), jnp.int32(0x3F800001)))
    maskf = jnp.where(bits >= tbits, one, zero)

    rk = lax.broadcasted_iota(jnp.int32, (_L, _L), 0)
    ck = lax.broadcasted_iota(jnp.int32, (_L, _L), 1)
    upper = jnp.where(rk <= ck, one, zero)
    incl = jnp.dot(maskf, upper, preferred_element_type=_F32, precision=_HI)
    rowcnt = incl[:, _L - 1:_L]
    ri = lax.broadcasted_iota(jnp.int32, (_R, _R), 0)
    ci = lax.broadcasted_iota(jnp.int32, (_R, _R), 1)
    strict = jnp.where(ci < ri, one, zero)
    off = jnp.dot(strict, rowcnt, preferred_element_type=_F32, precision=_HI)
    pos = off + incl - one

    flat = (lax.broadcasted_iota(jnp.int32, (_R, _L), 0) * _L
            + lax.broadcasted_iota(jnp.int32, (_R, _L), 1)).astype(_F32)

    ioc = lax.broadcasted_iota(jnp.int32, (_NC, 1), 0).astype(_F32)
    regx, regy = reg_ref[0, 0], reg_ref[0, 1]
    whc = [wh_ref[0, c] for c in range(4)]

    acc = jnp.zeros((_NC, 8), _F32)
    for c in range(_NCH):
        r0 = c * _CR
        sl = slice(r0, r0 + _CR)
        prow = pos[sl, :].reshape(1, _CW)
        mrow = maskf[sl, :].reshape(1, _CW)
        oht = jnp.where((prow == ioc) & (mrow > 0), one, zero)
        payload = jnp.concatenate(
            [ssup[sl, :].reshape(1, _CW), flat[sl, :].reshape(1, _CW),
             regx[sl, :].reshape(1, _CW), regy[sl, :].reshape(1, _CW),
             whc[0][sl, :].reshape(1, _CW), whc[1][sl, :].reshape(1, _CW),
             whc[2][sl, :].reshape(1, _CW), whc[3][sl, :].reshape(1, _CW)],
            axis=0)
        acc = acc + lax.dot_general(oht, payload, (((1,), (1,)), ((), ())),
                                    preferred_element_type=_F32, precision=_HI)

    cand_col, cidx_col = acc[:, 0:1], acc[:, 1:2]
    eye = jnp.where(
        lax.broadcasted_iota(jnp.int32, (_NC, _NC), 0)
        == lax.broadcasted_iota(jnp.int32, (_NC, _NC), 1), one, zero)
    cand_row = lax.dot_general(cand_col, eye, (((0,), (0,)), ((), ())),
                               preferred_element_type=_F32, precision=_HI)

    ii = lax.broadcasted_iota(jnp.int32, (_NC, _NC), 0)
    jj = lax.broadcasted_iota(jnp.int32, (_NC, _NC), 1)
    beats = (cand_row > cand_col) | ((cand_row == cand_col) & (jj < ii))
    rank = jnp.sum(jnp.where(beats, one, zero), axis=1, keepdims=True)
    ior = lax.broadcasted_iota(jnp.int32, (1, _NC), 1).astype(_F32)
    perm = jnp.where(rank == ior, one, zero)

    def srt(col):
        return lax.dot_general(perm, col, (((0,), (0,)), ((), ())),
                               preferred_element_type=_F32, precision=_HI)

    sc = srt(cand_col)
    si = srt(cidx_col)
    rgx, rgy = srt(acc[:, 2:3]), srt(acc[:, 3:4])
    w0, w1 = srt(acc[:, 4:5]), srt(acc[:, 5:6])
    w2, w3 = srt(acc[:, 6:7]), srt(acc[:, 7:8])

    scores_ref[0] = sc[:_K, :]
    idx_ref[0] = si.astype(jnp.int32)
    ysb = jnp.floor(si[:_K, :] / _F32(_W))
    xsb = si[:_K, :] - ysb * _F32(_W)
    xa = xsb + rgx[:_K, :]
    ya = ysb + rgy[:_K, :]
    w0, w1, w2, w3 = w0[:_K, :], w1[:_K, :], w2[:_K, :], w3[:_K, :]
    tlbr_ref[0] = jnp.concatenate([xa - w0, ya - w1, xa + w2, ya + w3], axis=1)
    xs_ref[0] = xa
    ys_ref[0] = ya
    wh2_ref[0] = jnp.concatenate([w0 + w2, w1 + w3], axis=1)


def _norm_kernel(g_ref, node_ref, fmap_ref):
    g = g_ref[0][: _K, :]
    nd = g[:, 0:128]
    fm = g[:, 128:384]
    nn = jnp.sqrt(jnp.sum(nd * nd, axis=1, keepdims=True))
    node_ref[0] = nd / jnp.maximum(nn, _F32(1e-12))
    fn = jnp.sqrt(jnp.sum(fm * fm, axis=1, keepdims=True))
    fmap_ref[0] = fm / jnp.maximum(fn, _F32(1e-12))


def _sc_gather(table, idx):
    info = plsc.get_sparse_core_info()
    nw = info.num_cores * info.num_subcores
    nrows = idx.shape[0]
    bpw = nrows // nw
    mesh = plsc.VectorSubcoreMesh(core_axis_name="c", subcore_axis_name="s")

    @functools.partial(
        pl.kernel, mesh=mesh,
        out_type=jax.ShapeDtypeStruct((nrows, _D), jnp.float32),
        scratch_types=[
            pltpu.VMEM((bpw,), jnp.int32),
            pltpu.VMEM((bpw, _D), jnp.float32),
            pltpu.SemaphoreType.DMA,
        ],
    )
    def k(table_hbm, idx_hbm, out_hbm, idx_v, rows_v, sem):
        wid = lax.axis_index("s") * info.num_cores + lax.axis_index("c")
        base = wid * bpw
        pltpu.sync_copy(idx_hbm.at[pl.ds(base, bpw)], idx_v)
        pltpu.async_copy(table_hbm.at[idx_v], rows_v, sem).wait()
        pltpu.sync_copy(rows_v, out_hbm.at[pl.ds(base, bpw)])

    return k(table, idx)


def kernel(hm, reg, wh, node, fmap):
    ssup = pl.pallas_call(
        _nms_kernel,
        grid=(_B,),
        in_specs=[pl.BlockSpec((1, 1, _H, _W), lambda b: (b, 0, 0, 0))],
        out_specs=pl.BlockSpec((1, 1, _H, _W), lambda b: (b, 0, 0, 0)),
        out_shape=jax.ShapeDtypeStruct((_B, 1, _H, _W), jnp.float32),
    )(hm)

    sflat = ssup.reshape(_B, 1, _R, _L)
    rflat = reg.reshape(_B, 2, _R, _L)
    wflat = wh.reshape(_B, 4, _R, _L)

    scores, idx512, tlbr, xs, ys, wh2 = pl.pallas_call(
        _topk_kernel,
        grid=(_B,),
        in_specs=[
            pl.BlockSpec((1, 1, _R, _L), lambda b: (b, 0, 0, 0)),
            pl.BlockSpec((1, 2, _R, _L), lambda b: (b, 0, 0, 0)),
            pl.BlockSpec((1, 4, _R, _L), lambda b: (b, 0, 0, 0)),
        ],
        out_specs=[
            pl.BlockSpec((1, _K, 1), lambda b: (b, 0, 0)),
            pl.BlockSpec((1, _NC, 1), lambda b: (b, 0, 0)),
            pl.BlockSpec((1, _K, 4), lambda b: (b, 0, 0)),
            pl.BlockSpec((1, _K, 1), lambda b: (b, 0, 0)),
            pl.BlockSpec((1, _K, 1), lambda b: (b, 0, 0)),
            pl.BlockSpec((1, _K, 2), lambda b: (b, 0, 0)),
        ],
        out_shape=[
            jax.ShapeDtypeStruct((_B, _K, 1), jnp.float32),
            jax.ShapeDtypeStruct((_B, _NC, 1), jnp.int32),
            jax.ShapeDtypeStruct((_B, _K, 4), jnp.float32),
            jax.ShapeDtypeStruct((_B, _K, 1), jnp.float32),
            jax.ShapeDtypeStruct((_B, _K, 1), jnp.float32),
            jax.ShapeDtypeStruct((_B, _K, 2), jnp.float32),
        ],
    )(sflat, rflat, wflat)

    table = jnp.concatenate([node, fmap], axis=1)
    table = table.transpose(0, 2, 3, 1).reshape(_B * _HW, _D)
    gidx = (jnp.arange(_B, dtype=jnp.int32)[:, None] * _HW
            + idx512.reshape(_B, _NC)).reshape(_B * _NC)
    gathered = _sc_gather(table, gidx).reshape(_B, _NC, _D)

    node_feat, fmap_feat = pl.pallas_call(
        _norm_kernel,
        grid=(_B,),
        in_specs=[pl.BlockSpec((1, _NC, _D), lambda b: (b, 0, 0))],
        out_specs=[
            pl.BlockSpec((1, _K, 128), lambda b: (b, 0, 0)),
            pl.BlockSpec((1, _K, 256), lambda b: (b, 0, 0)),
        ],
        out_shape=[
            jax.ShapeDtypeStruct((_B, _K, 128), jnp.float32),
            jax.ShapeDtypeStruct((_B, _K, 256), jnp.float32),
        ],
    )(gathered)

    clses = jnp.zeros((_B, _K, 1), jnp.int32)
    return (tlbr, scores, clses, xs, ys, wh2, node_feat, fmap_feat)
